# baseline (speedup 1.0000x reference)
import functools

import jax
import jax.numpy as jnp
from jax import lax
from jax.experimental import pallas as pl
from jax.experimental.pallas import tpu as pltpu
from jax.experimental.pallas import tpu_sc as plsc

_NUM_FIELDS = 26
_VOCAB = 100000
_EMB_DIM = 32
_BATCH = 4096

_NC = 2
_NS = 16
_NW = _NC * _NS

_ROWS = _NUM_FIELDS * _EMB_DIM
_CHUNKS = (384, 448)

_NT = 100096 // 128
_GWORDS = _NT * 1024
_SLICE = 799872
_CT = 782

_mesh = plsc.VectorSubcoreMesh(core_axis_name="c", subcore_axis_name="s")


def _relabel_body(in_ref, out_ref):
    blk = in_ref[...]
    out_ref[...] = (
        blk.reshape(8, _CT, 128).swapaxes(0, 1).reshape(_CT * 8, 128)
    )


def _make_relabel(group0, ngroups):
    return pl.pallas_call(
        _relabel_body,
        out_shape=jax.ShapeDtypeStruct((ngroups * _NT * 8, 128), jnp.float32),
        grid=(ngroups,),
        in_specs=[pl.BlockSpec((8, _CT * 128), lambda g: (g + group0, 0))],
        out_specs=pl.BlockSpec((_CT * 8, 128), lambda g: (g, 0)),
    )


def _make_gather(row0, nrows):
    rpw = nrows // _NW

    @functools.partial(
        pl.kernel,
        out_type=jax.ShapeDtypeStruct((nrows * _BATCH,), jnp.float32),
        mesh=_mesh,
        scratch_types=[
            pltpu.VMEM((2 * _BATCH,), jnp.int32),
            pltpu.VMEM((rpw * _BATCH,), jnp.float32),
            pltpu.SemaphoreType.DMA,
        ],
        compiler_params=pltpu.CompilerParams(use_tc_tiling_on_sc=False),
    )
    def _gather_chunk(ids_hbm, flat_hbm, out_hbm, offs_v, rows_v, sem):
        wid = lax.axis_index("s") * _NC + lax.axis_index("c")
        r0 = wid * rpw
        f0 = (row0 + r0) // _EMB_DIM
        f1 = jnp.minimum(f0 + 1, _NUM_FIELDS - 1)

        pltpu.sync_copy(ids_hbm.at[pl.ds(f0 * _BATCH, _BATCH)],
                        offs_v.at[pl.ds(0, _BATCH)])
        pltpu.sync_copy(ids_hbm.at[pl.ds(f1 * _BATCH, _BATCH)],
                        offs_v.at[pl.ds(_BATCH, _BATCH)])

        def to_offs(p, _):
            sl = pl.ds(pl.multiple_of(p * 16, 16), 16)
            v = offs_v[sl]
            offs_v[sl] = ((v >> 7) << 10) | (v & 127)
            return 0

        lax.fori_loop(0, 2 * _BATCH // 16, to_offs, 0)

        def copy_q(q):
            r = r0 + q
            lf = (row0 + r) // _EMB_DIM - f0
            base = pl.multiple_of((r // 8) * _GWORDS + (r % 8) * 128, 128)
            return pltpu.make_async_copy(
                flat_hbm.at[pl.ds(base, _SLICE)]
                        .at[offs_v.at[pl.ds(lf * _BATCH, _BATCH)]],
                rows_v.at[pl.ds(q * _BATCH, _BATCH)],
                sem,
            )

        def fire(q, _):
            copy_q(q).start()
            return 0

        def drain(q, _):
            copy_q(q).wait()
            return 0

        lax.fori_loop(0, rpw, fire, 0)
        lax.fori_loop(0, rpw, drain, 0)

        pltpu.sync_copy(rows_v, out_hbm.at[pl.ds(r0 * _BATCH, rpw * _BATCH)])

    return _gather_chunk


def _chunk_row0s():
    acc, out = 0, []
    for n in _CHUNKS:
        out.append(acc)
        acc += n
    return tuple(out)


_ROW0S = _chunk_row0s()
_relabels = [_make_relabel(r0 // 8, n // 8) for r0, n in zip(_ROW0S, _CHUNKS)]
_gathers = [_make_gather(r0, n) for r0, n in zip(_ROW0S, _CHUNKS)]


def kernel(x, tables):
    ids = x.T.reshape(_NUM_FIELDS * _BATCH)
    tab2 = tables.transpose(0, 2, 1).reshape(_ROWS, _VOCAB)
    outs = []
    for k in range(len(_CHUNKS)):
        nflat = (_CHUNKS[k] // 8) * _NT * 8 * 128
        flat = _relabels[k](tab2).reshape(nflat)
        outs.append(_gathers[k](ids, flat))
    out = jnp.concatenate(outs)
    return out.reshape(_NUM_FIELDS, _EMB_DIM, _BATCH).transpose(2, 0, 1)


# speedup vs baseline: 1.0219x; 1.0219x over previous
import functools

import jax
import jax.numpy as jnp
from jax import lax
from jax.experimental import pallas as pl
from jax.experimental.pallas import tpu as pltpu
from jax.experimental.pallas import tpu_sc as plsc

_NUM_FIELDS = 26
_VOCAB = 100000
_EMB_DIM = 32
_BATCH = 4096

_NC = 2
_NS = 16
_NW = _NC * _NS

_ROWS = _NUM_FIELDS * _EMB_DIM
_CHUNKS = (416, 416)

_NT = 100096 // 128
_GWORDS = _NT * 1024
_SLICE = 799872
_CT = 782

_mesh = plsc.VectorSubcoreMesh(core_axis_name="c", subcore_axis_name="s")


def _relabel_body(in_ref, out_ref):
    blk = in_ref[...]
    out_ref[...] = (
        blk.reshape(8, _CT, 128).swapaxes(0, 1).reshape(_CT * 8, 128)
    )


def _make_relabel(group0, ngroups):
    return pl.pallas_call(
        _relabel_body,
        out_shape=jax.ShapeDtypeStruct((ngroups * _NT * 8, 128), jnp.float32),
        grid=(ngroups,),
        in_specs=[pl.BlockSpec((8, _CT * 128), lambda g: (g + group0, 0))],
        out_specs=pl.BlockSpec((_CT * 8, 128), lambda g: (g, 0)),
    )


def _make_gather(row0, nrows):
    rpw = nrows // _NW

    @functools.partial(
        pl.kernel,
        out_type=jax.ShapeDtypeStruct((nrows * _BATCH,), jnp.float32),
        mesh=_mesh,
        scratch_types=[
            pltpu.VMEM((2 * _BATCH,), jnp.int32),
            pltpu.VMEM((rpw * _BATCH,), jnp.float32),
            pltpu.SemaphoreType.DMA,
        ],
        compiler_params=pltpu.CompilerParams(use_tc_tiling_on_sc=False),
    )
    def _gather_chunk(ids_hbm, flat_hbm, out_hbm, offs_v, rows_v, sem):
        wid = lax.axis_index("s") * _NC + lax.axis_index("c")
        r0 = wid * rpw
        f0 = (row0 + r0) // _EMB_DIM
        f1 = jnp.minimum(f0 + 1, _NUM_FIELDS - 1)

        pltpu.sync_copy(ids_hbm.at[pl.ds(f0 * _BATCH, _BATCH)],
                        offs_v.at[pl.ds(0, _BATCH)])
        pltpu.sync_copy(ids_hbm.at[pl.ds(f1 * _BATCH, _BATCH)],
                        offs_v.at[pl.ds(_BATCH, _BATCH)])

        def to_offs(p, _):
            sl = pl.ds(pl.multiple_of(p * 16, 16), 16)
            v = offs_v[sl]
            offs_v[sl] = ((v >> 7) << 10) | (v & 127)
            return 0

        lax.fori_loop(0, 2 * _BATCH // 16, to_offs, 0)

        def copy_q(q):
            r = r0 + q
            lf = (row0 + r) // _EMB_DIM - f0
            base = pl.multiple_of((r // 8) * _GWORDS + (r % 8) * 128, 128)
            return pltpu.make_async_copy(
                flat_hbm.at[pl.ds(base, _SLICE)]
                        .at[offs_v.at[pl.ds(lf * _BATCH, _BATCH)]],
                rows_v.at[pl.ds(q * _BATCH, _BATCH)],
                sem,
            )

        def fire(q, _):
            copy_q(q).start()
            return 0

        def drain(q, _):
            copy_q(q).wait()
            return 0

        lax.fori_loop(0, rpw, fire, 0)
        lax.fori_loop(0, rpw, drain, 0)

        pltpu.sync_copy(rows_v, out_hbm.at[pl.ds(r0 * _BATCH, rpw * _BATCH)])

    return _gather_chunk


def _chunk_row0s():
    acc, out = 0, []
    for n in _CHUNKS:
        out.append(acc)
        acc += n
    return tuple(out)


_ROW0S = _chunk_row0s()
_relabels = [_make_relabel(r0 // 8, n // 8) for r0, n in zip(_ROW0S, _CHUNKS)]
_gathers = [_make_gather(r0, n) for r0, n in zip(_ROW0S, _CHUNKS)]


def kernel(x, tables):
    ids = x.T.reshape(_NUM_FIELDS * _BATCH)
    tab2 = tables.transpose(0, 2, 1).reshape(_ROWS, _VOCAB)
    outs = []
    for k in range(len(_CHUNKS)):
        nflat = (_CHUNKS[k] // 8) * _NT * 8 * 128
        flat = _relabels[k](tab2).reshape(nflat)
        outs.append(_gathers[k](ids, flat))
    out = jnp.concatenate(outs)
    return out.reshape(_NUM_FIELDS, _EMB_DIM, _BATCH).transpose(2, 0, 1)
